# baseline (speedup 1.0000x reference)
import functools

import jax
import jax.numpy as jnp
from jax import lax
from jax.experimental import pallas as pl
from jax.experimental.pallas import tpu as pltpu
from jax.experimental.pallas import tpu_sc as plsc

IN_FEATURES = 128
C0 = 20000
C1 = 100000
C2 = 1000000
D1 = 64
D2 = 32

NC = 2
NS = 16
NW = NC * NS
LANES = 16

CH = 128


def _sc_gather(idx_flat, head_weight, tail0_emb, tail1_emb):
    B = idx_flat.shape[0]
    assert B % (NW * CH) == 0
    b_per_w = B // NW
    n_chunks = b_per_w // CH

    mesh = plsc.VectorSubcoreMesh(core_axis_name="c", subcore_axis_name="s")

    @functools.partial(
        pl.kernel,
        mesh=mesh,
        compiler_params=pltpu.CompilerParams(use_tc_tiling_on_sc=False),
        out_type=(
            jax.ShapeDtypeStruct((B, IN_FEATURES), jnp.float32),
            jax.ShapeDtypeStruct((B, D1), jnp.float32),
            jax.ShapeDtypeStruct((B, D2), jnp.float32),
        ),
        scratch_types=(
            pltpu.VMEM((CH,), jnp.int32),
            pltpu.VMEM((CH,), jnp.int32),
            pltpu.VMEM((CH,), jnp.int32),
            pltpu.VMEM((CH,), jnp.int32),
            pltpu.VMEM((CH, IN_FEATURES), jnp.float32),
            pltpu.VMEM((CH, D1), jnp.float32),
            pltpu.VMEM((CH, D2), jnp.float32),
            pltpu.SemaphoreType.DMA,
        ),
    )
    def gather_kernel(idx_hbm, head_hbm, t0_hbm, t1_hbm,
                      g0_hbm, g1_hbm, g2_hbm,
                      idx_v, r0_v, r1_v, r2_v, b0_v, b1_v, b2_v, sem):
        wid = lax.axis_index("s") * NC + lax.axis_index("c")
        base = wid * b_per_w

        def chunk_body(c, carry):
            off = base + c * CH
            pltpu.sync_copy(idx_hbm.at[pl.ds(off, CH)], idx_v)

            def vec_body(j, carry2):
                sl = pl.ds(j * LANES, LANES)
                v = idx_v[sl]
                r0_v[sl] = jnp.minimum(jnp.maximum(v, 0), C0 - 1)
                r1_v[sl] = jnp.minimum(jnp.maximum(v - C0, 0), C1 - C0 - 1)
                r2_v[sl] = jnp.minimum(jnp.maximum(v - C1, 0), C2 - C1 - 1)
                return carry2

            lax.fori_loop(0, CH // LANES, vec_body, 0)

            c0 = pltpu.async_copy(head_hbm.at[r0_v], b0_v, sem)
            c1 = pltpu.async_copy(t0_hbm.at[r1_v], b1_v, sem)
            c2 = pltpu.async_copy(t1_hbm.at[r2_v], b2_v, sem)
            c0.wait()
            c1.wait()
            c2.wait()

            pltpu.sync_copy(b0_v, g0_hbm.at[pl.ds(off, CH)])
            pltpu.sync_copy(b1_v, g1_hbm.at[pl.ds(off, CH)])
            pltpu.sync_copy(b2_v, g2_hbm.at[pl.ds(off, CH)])
            return carry

        lax.fori_loop(0, n_chunks, chunk_body, 0)

    return gather_kernel(idx_flat, head_weight, tail0_emb, tail1_emb)


def _tc_combine(idx3, g0, g1, g2, w0t, w1t, bt):
    B = g0.shape[0]
    nb = B // bt

    def body(idx_ref, g0_ref, g1_ref, g2_ref, w0t_ref, w1t_ref, out_ref):
        idx = idx_ref[...]
        m0 = (idx >= 0) & (idx < C0)
        m1 = (idx >= C0) & (idx < C1)
        m2 = (idx >= C1) & (idx < C2)
        p1 = jnp.dot(g1_ref[...], w0t_ref[...],
                     preferred_element_type=jnp.float32)
        p2 = jnp.dot(g2_ref[...], w1t_ref[...],
                     preferred_element_type=jnp.float32)
        zero = jnp.zeros_like(out_ref)
        out_ref[...] = (
            jnp.where(m0, g0_ref[...], zero)
            + jnp.where(m1, p1, zero)
            + jnp.where(m2, p2, zero)
        )

    return pl.pallas_call(
        body,
        grid=(nb,),
        in_specs=[
            pl.BlockSpec((bt, 1), lambda i: (i, 0)),
            pl.BlockSpec((bt, IN_FEATURES), lambda i: (i, 0)),
            pl.BlockSpec((bt, D1), lambda i: (i, 0)),
            pl.BlockSpec((bt, D2), lambda i: (i, 0)),
            pl.BlockSpec((D1, IN_FEATURES), lambda i: (0, 0)),
            pl.BlockSpec((D2, IN_FEATURES), lambda i: (0, 0)),
        ],
        out_specs=pl.BlockSpec((bt, IN_FEATURES), lambda i: (i, 0)),
        out_shape=jax.ShapeDtypeStruct((B, IN_FEATURES), jnp.float32),
    )(idx3, g0, g1, g2, w0t, w1t)


def kernel(myinput, head_weight, tail0_emb, tail0_w, tail1_emb, tail1_w):
    Bm, L = myinput.shape
    B = Bm * L
    idx_flat = myinput.reshape(B)

    g0, g1, g2 = _sc_gather(idx_flat, head_weight, tail0_emb, tail1_emb)

    bt = 4096
    idx_col = idx_flat.reshape(B, 1)
    out = _tc_combine(idx_col, g0, g1, g2, tail0_w.T, tail1_w.T, bt)
    return out.reshape(Bm, L, IN_FEATURES)
